# baseline (speedup 1.0000x reference)
import functools

import jax
import jax.numpy as jnp
from jax import lax
from jax.experimental import pallas as pl
from jax.experimental.pallas import tpu as pltpu
from jax.experimental.pallas import tpu_sc as plsc

NC = 2
NS = 16
NW = NC * NS
L = 16
U = 8
UB = 16
B = 128
CB = 22
K = 71


def _edge_mlp(table_v, w0, w1, w2, b, scaler, penalty, s, d, c):
    xs = plsc.load_gather(table_v, [s])
    xd = plsc.load_gather(table_v, [d])
    lin = w0 * xs + w1 * xd + w2 * c + b
    lr = jnp.where(lin >= 0.0, lin, lin * jnp.float32(0.01))
    raw = lr + c * penalty
    att = 1.0 / (1.0 + jnp.exp(-raw))
    return xs * att * scaler


def _fused_body(N, nf_hbm, edge_hbm, cm_hbm, pars_hbm, msg_hbm, part_hbm,
                table_v, pars_v, eb1_v, cm1_v, msg1_v):
    wid = lax.axis_index("s") * NC + lax.axis_index("c")
    pltpu.sync_copy(nf_hbm, table_v)
    pltpu.sync_copy(pars_hbm, pars_v)
    w0 = pars_v[pl.ds(0 * L, L)]
    w1 = pars_v[pl.ds(1 * L, L)]
    w2 = pars_v[pl.ds(2 * L, L)]
    b = pars_v[pl.ds(3 * L, L)]
    scaler = pars_v[pl.ds(4 * L, L)]
    penalty = pars_v[pl.ds(5 * L, L)]
    C = CB * B

    def inner_a(eb_v, cm_v, msg_v):
        @plsc.parallel_loop(0, C, L, unroll=U)
        def _(o):
            blk = o // B
            sub = o % B
            s = eb_v[blk, 0, pl.ds(sub, L)]
            d = eb_v[blk, 1, pl.ds(sub, L)]
            c = cm_v[pl.ds(o, L)]
            msg_v[pl.ds(o, L)] = _edge_mlp(
                table_v, w0, w1, w2, b, scaler, penalty, s, d, c)

    pltpu.emit_pipeline(
        inner_a,
        grid=(K,),
        in_specs=[
            pl.BlockSpec((CB, 2, B), lambda i: (wid * K + i, 0, 0)),
            pl.BlockSpec((C,), lambda i: (wid * K + i,)),
        ],
        out_specs=[pl.BlockSpec((C,), lambda i: (wid * K + i,))],
    )(edge_hbm, cm_hbm, msg_hbm)

    @pl.when(wid >= NW // 2)
    def _():
        my_blk = NW * K * CB + (wid - NW // 2)
        pltpu.sync_copy(edge_hbm.at[my_blk], eb1_v)
        pltpu.sync_copy(cm_hbm.at[pl.ds(my_blk * B, B)], cm1_v)
        for g in range(B // L):
            o = g * L
            s = eb1_v[0, pl.ds(o, L)]
            d = eb1_v[1, pl.ds(o, L)]
            c = cm1_v[pl.ds(o, L)]
            msg1_v[pl.ds(o, L)] = _edge_mlp(
                table_v, w0, w1, w2, b, scaler, penalty, s, d, c)
        pltpu.sync_copy(msg1_v, msg_hbm.at[pl.ds(my_blk * B, B)])

    @plsc.parallel_loop(0, N, L)
    def _(o):
        table_v[pl.ds(o, L)] = jnp.zeros((L,), jnp.float32)

    def inner_b(eb_v, msg_v):
        @plsc.parallel_loop(0, C, L, unroll=UB)
        def _(o):
            blk = o // B
            sub = o % B
            d = eb_v[blk, 1, pl.ds(sub, L)]
            m = msg_v[pl.ds(o, L)]
            plsc.addupdate_scatter(table_v, [d], m)

    pltpu.emit_pipeline(
        inner_b,
        grid=(K,),
        in_specs=[
            pl.BlockSpec((CB, 2, B), lambda i: (wid * K + i, 0, 0)),
            pl.BlockSpec((C,), lambda i: (wid * K + i,)),
        ],
        out_specs=[],
    )(edge_hbm, msg_hbm)

    @pl.when(wid >= NW // 2)
    def _():
        my_blk = NW * K * CB + (wid - NW // 2)
        pltpu.sync_copy(edge_hbm.at[my_blk], eb1_v)
        pltpu.sync_copy(msg_hbm.at[pl.ds(my_blk * B, B)], msg1_v)
        for g in range(B // L):
            o = g * L
            d = eb1_v[1, pl.ds(o, L)]
            m = msg1_v[pl.ds(o, L)]
            plsc.addupdate_scatter(table_v, [d], m)

    pltpu.sync_copy(table_v, part_hbm.at[wid])


def _reduce_body(part_ref, out_ref):
    out_ref[...] = jnp.sum(part_ref[...], axis=0)


def kernel(node_features, edge_index, cycle_mask, min_sum_scaler, att_w,
           att_b, cycle_penalty):
    N = node_features.shape[0]
    E = edge_index.shape[1]
    assert E % B == 0 and N % L == 0
    nb_tot = E // B
    assert NW * CB * K + NW // 2 == nb_tot

    vals = jnp.concatenate([
        att_w.reshape(3), att_b.reshape(1),
        min_sum_scaler.reshape(1), cycle_penalty.reshape(1),
    ]).astype(jnp.float32)
    pars96 = jnp.broadcast_to(vals[:, None], (6, L)).reshape(6 * L)

    mesh = plsc.VectorSubcoreMesh(
        core_axis_name="c", subcore_axis_name="s",
        num_cores=NC, num_subcores=NS)
    sc_params = pltpu.CompilerParams(needs_layout_passes=False)

    edge3 = edge_index.reshape(2, nb_tot, B).transpose(1, 0, 2)

    _, part = pl.kernel(
        functools.partial(_fused_body, N),
        out_type=[
            jax.ShapeDtypeStruct((E,), jnp.float32),
            jax.ShapeDtypeStruct((NW, N), jnp.float32),
        ],
        mesh=mesh, compiler_params=sc_params,
        scratch_types=[
            pltpu.VMEM((N,), jnp.float32),
            pltpu.VMEM((6 * L,), jnp.float32),
            pltpu.VMEM((2, B), jnp.int32),
            pltpu.VMEM((B,), jnp.float32),
            pltpu.VMEM((B,), jnp.float32),
        ],
    )(node_features, edge3, cycle_mask, pars96)

    out = pl.pallas_call(
        _reduce_body,
        out_shape=jax.ShapeDtypeStruct((N,), jnp.float32),
    )(part)
    return out


# speedup vs baseline: 1.0064x; 1.0064x over previous
import functools

import jax
import jax.numpy as jnp
from jax import lax
from jax.experimental import pallas as pl
from jax.experimental.pallas import tpu as pltpu
from jax.experimental.pallas import tpu_sc as plsc

NC = 2
NS = 16
NW = NC * NS
L = 16
U = 11
UB = 16
B = 128
CB = 22
K = 71


def _edge_mlp(table_v, w0, w1, w2, b, scaler, penalty, s, d, c):
    xs = plsc.load_gather(table_v, [s])
    xd = plsc.load_gather(table_v, [d])
    lin = w0 * xs + w1 * xd + w2 * c + b
    lr = jnp.where(lin >= 0.0, lin, lin * jnp.float32(0.01))
    raw = lr + c * penalty
    att = 1.0 / (1.0 + jnp.exp(-raw))
    return xs * att * scaler


def _fused_body(N, nf_hbm, edge_hbm, cm_hbm, pars_hbm, msg_hbm, part_hbm,
                table_v, pars_v, eb1_v, cm1_v, msg1_v):
    wid = lax.axis_index("s") * NC + lax.axis_index("c")
    pltpu.sync_copy(nf_hbm, table_v)
    pltpu.sync_copy(pars_hbm, pars_v)
    w0 = pars_v[pl.ds(0 * L, L)]
    w1 = pars_v[pl.ds(1 * L, L)]
    w2 = pars_v[pl.ds(2 * L, L)]
    b = pars_v[pl.ds(3 * L, L)]
    scaler = pars_v[pl.ds(4 * L, L)]
    penalty = pars_v[pl.ds(5 * L, L)]
    C = CB * B

    def inner_a(eb_v, cm_v, msg_v):
        @plsc.parallel_loop(0, C, L, unroll=U)
        def _(o):
            blk = o // B
            sub = o % B
            s = eb_v[blk, 0, pl.ds(sub, L)]
            d = eb_v[blk, 1, pl.ds(sub, L)]
            c = cm_v[pl.ds(o, L)]
            msg_v[pl.ds(o, L)] = _edge_mlp(
                table_v, w0, w1, w2, b, scaler, penalty, s, d, c)

    pltpu.emit_pipeline(
        inner_a,
        grid=(K,),
        in_specs=[
            pl.BlockSpec((CB, 2, B), lambda i: (wid * K + i, 0, 0)),
            pl.BlockSpec((C,), lambda i: (wid * K + i,)),
        ],
        out_specs=[pl.BlockSpec((C,), lambda i: (wid * K + i,))],
    )(edge_hbm, cm_hbm, msg_hbm)

    @pl.when(wid >= NW // 2)
    def _():
        my_blk = NW * K * CB + (wid - NW // 2)
        pltpu.sync_copy(edge_hbm.at[my_blk], eb1_v)
        pltpu.sync_copy(cm_hbm.at[pl.ds(my_blk * B, B)], cm1_v)
        for g in range(B // L):
            o = g * L
            s = eb1_v[0, pl.ds(o, L)]
            d = eb1_v[1, pl.ds(o, L)]
            c = cm1_v[pl.ds(o, L)]
            msg1_v[pl.ds(o, L)] = _edge_mlp(
                table_v, w0, w1, w2, b, scaler, penalty, s, d, c)
        pltpu.sync_copy(msg1_v, msg_hbm.at[pl.ds(my_blk * B, B)])

    @plsc.parallel_loop(0, N, L)
    def _(o):
        table_v[pl.ds(o, L)] = jnp.zeros((L,), jnp.float32)

    def inner_b(eb_v, msg_v):
        @plsc.parallel_loop(0, C, L, unroll=UB)
        def _(o):
            blk = o // B
            sub = o % B
            d = eb_v[blk, 1, pl.ds(sub, L)]
            m = msg_v[pl.ds(o, L)]
            plsc.addupdate_scatter(table_v, [d], m)

    pltpu.emit_pipeline(
        inner_b,
        grid=(K,),
        in_specs=[
            pl.BlockSpec((CB, 2, B), lambda i: (wid * K + i, 0, 0)),
            pl.BlockSpec((C,), lambda i: (wid * K + i,)),
        ],
        out_specs=[],
    )(edge_hbm, msg_hbm)

    @pl.when(wid >= NW // 2)
    def _():
        my_blk = NW * K * CB + (wid - NW // 2)
        pltpu.sync_copy(edge_hbm.at[my_blk], eb1_v)
        pltpu.sync_copy(msg_hbm.at[pl.ds(my_blk * B, B)], msg1_v)
        for g in range(B // L):
            o = g * L
            d = eb1_v[1, pl.ds(o, L)]
            m = msg1_v[pl.ds(o, L)]
            plsc.addupdate_scatter(table_v, [d], m)

    pltpu.sync_copy(table_v, part_hbm.at[wid])


def _reduce_body(part_ref, out_ref):
    out_ref[...] = jnp.sum(part_ref[...], axis=0)


def kernel(node_features, edge_index, cycle_mask, min_sum_scaler, att_w,
           att_b, cycle_penalty):
    N = node_features.shape[0]
    E = edge_index.shape[1]
    assert E % B == 0 and N % L == 0
    nb_tot = E // B
    assert NW * CB * K + NW // 2 == nb_tot

    vals = jnp.concatenate([
        att_w.reshape(3), att_b.reshape(1),
        min_sum_scaler.reshape(1), cycle_penalty.reshape(1),
    ]).astype(jnp.float32)
    pars96 = jnp.broadcast_to(vals[:, None], (6, L)).reshape(6 * L)

    mesh = plsc.VectorSubcoreMesh(
        core_axis_name="c", subcore_axis_name="s",
        num_cores=NC, num_subcores=NS)
    sc_params = pltpu.CompilerParams(needs_layout_passes=False)

    edge3 = edge_index.reshape(2, nb_tot, B).transpose(1, 0, 2)

    _, part = pl.kernel(
        functools.partial(_fused_body, N),
        out_type=[
            jax.ShapeDtypeStruct((E,), jnp.float32),
            jax.ShapeDtypeStruct((NW, N), jnp.float32),
        ],
        mesh=mesh, compiler_params=sc_params,
        scratch_types=[
            pltpu.VMEM((N,), jnp.float32),
            pltpu.VMEM((6 * L,), jnp.float32),
            pltpu.VMEM((2, B), jnp.int32),
            pltpu.VMEM((B,), jnp.float32),
            pltpu.VMEM((B,), jnp.float32),
        ],
    )(node_features, edge3, cycle_mask, pars96)

    out = pl.pallas_call(
        _reduce_body,
        out_shape=jax.ShapeDtypeStruct((N,), jnp.float32),
    )(part)
    return out
